# baseline (speedup 1.0000x reference)
import functools

import jax
import jax.numpy as jnp
import numpy as np
from jax.experimental import pallas as pl


def _make_pe(L, d):
    position = np.arange(L)[:, None].astype(np.float32)
    div_term = np.exp(np.arange(0, d, 2).astype(np.float32) * -(np.log(10000.0) / d))
    pe = np.zeros((L, d), dtype=np.float32)
    pe[:, 0::2] = np.sin(position * div_term)
    pe[:, 1::2] = np.cos(position * div_term)
    return jnp.asarray(pe)


def _pred_body(h_ref, soc_ref, out_ref):
    out_ref[...] = jax.lax.dot_general(
        h_ref[...], soc_ref[...], (((1,), (1,)), ((), ())),
        preferred_element_type=jnp.float32)


def _pred_matmul(h, soc_tab):
    B, d = h.shape
    N = soc_tab.shape[0]
    NP = ((N + 1023) // 1024) * 1024
    soc_p = jnp.pad(soc_tab, ((0, NP - N), (0, 0)))
    TN = 1024
    out = pl.pallas_call(
        _pred_body,
        grid=(NP // TN,),
        in_specs=[
            pl.BlockSpec((B, d), lambda i: (0, 0)),
            pl.BlockSpec((TN, d), lambda i: (i, 0)),
        ],
        out_specs=pl.BlockSpec((B, TN), lambda i: (0, i)),
        out_shape=jax.ShapeDtypeStruct((B, NP), jnp.float32),
    )(h, soc_p)
    return out[:, :N]


def kernel(social_hypergraph_list, cascade_hypergraph, examples, masks, lambda_u,
           rel_emb, rel_theta, rel_bias, cas_emb, cas_theta, cas_bias,
           infl_emb, co_attn_wts, f1_w1, f1_b1, f1_w2, f1_b2, soc_tab, tmp_tab):
    N = rel_emb.shape[0]
    d = rel_emb.shape[1]

    def hconv(x, hg, theta, bias):
        xt = x @ theta
        ni = hg[0]
        ei = hg[1]
        Ddeg = jax.ops.segment_sum(jnp.ones(ni.shape, xt.dtype), ni, num_segments=N)
        Bdeg = jax.ops.segment_sum(jnp.ones(ei.shape, xt.dtype), ei, num_segments=N)
        Dinv = jnp.where(Ddeg > 0, 1.0 / Ddeg, 0.0)
        Binv = jnp.where(Bdeg > 0, 1.0 / Bdeg, 0.0)
        m = jax.ops.segment_sum(xt[ni], ei, num_segments=N) * Binv[:, None]
        out = jax.ops.segment_sum(m[ei], ni, num_segments=N) * Dinv[:, None]
        return out + bias

    n_hg = social_hypergraph_list.shape[0]
    embs = [hconv(rel_emb, social_hypergraph_list[i], rel_theta, rel_bias) for i in range(n_hg)]

    ws = [jnp.tanh(e @ f1_w1.T + f1_b1) @ f1_w2.T + f1_b2 for e in embs]
    temp = jax.nn.softmax(jnp.concatenate(ws, axis=1), axis=1)
    user_social = embs[0] * temp[:, 0:1]
    for i in range(1, n_hg):
        user_social = user_social + embs[0] * temp[:, 0:1]

    mf = masks.astype(jnp.float32)
    sender_social = soc_tab[examples] * mf[..., None]

    user_temporal = hconv(cas_emb, cascade_hypergraph, cas_theta, cas_bias)
    sender_temporal = tmp_tab[examples] * mf[..., None]

    L = examples.shape[1]
    pe = _make_pe(L, d)
    infl = infl_emb[examples] * mf[..., None]
    st = sender_temporal + pe[None, :, :]
    st = st + infl
    attn_act = jnp.tanh(jnp.sum(jnp.tensordot(sender_social, co_attn_wts, axes=([2], [0])) * st, axis=2))
    alpha = jax.nn.softmax(attn_act, axis=1)
    h = jnp.sum(st * alpha[..., None], axis=1)

    pred = _pred_matmul(h, soc_tab)
    user_loss = 0.5 * lambda_u * jnp.mean(
        jnp.sum(jnp.square(user_temporal - tmp_tab), axis=1)
        + jnp.sum(jnp.square(user_social - soc_tab), axis=1))
    return (pred, co_attn_wts, user_loss)
